# baseline (speedup 1.0000x reference)
import functools

import jax
import jax.numpy as jnp
from jax import lax
from jax.experimental import pallas as pl
from jax.experimental.pallas import tpu as pltpu
from jax.experimental.pallas import tpu_sc as plsc

NC = 2
NS = 16
NW = NC * NS
CH = 128
UN = 2


@functools.lru_cache(maxsize=None)
def _make_seg_sum(n_pad: int, w: int, e_pad: int):
  assert e_pad % (CH * NW * UN) == 0 and n_pad % (NS * 8) == 0
  rows_per_tile = n_pad // NS
  j_max = e_pad // (CH * NW)

  mesh = plsc.VectorSubcoreMesh(
      core_axis_name="c", subcore_axis_name="s",
      num_cores=NC, num_subcores=NS)

  def body(p_hbm, src_hbm, dst_hbm, z_hbm, g_out, acc_sh,
           src_all, dst_all, rows_a, rows_b, sem_a, sem_b):
    cid = lax.axis_index("c")
    sid = lax.axis_index("s")
    wid = sid * NC + cid
    base = sid * rows_per_tile
    cbase = wid * j_max

    pltpu.sync_copy(src_hbm.at[pl.ds(cbase, j_max)], src_all)
    pltpu.sync_copy(dst_hbm.at[pl.ds(cbase, j_max)], dst_all)
    pltpu.sync_copy(z_hbm, acc_sh.at[pl.ds(base, rows_per_tile)])
    plsc.subcore_barrier()

    def group(g, carry):
      j0 = g * UN
      cp_a = pltpu.async_copy(p_hbm.at[src_all.at[j0, 0]], rows_a, sem_a)
      cp_b = pltpu.async_copy(p_hbm.at[src_all.at[j0 + 1, 0]], rows_b, sem_b)
      cp_a.wait()
      pltpu.sync_copy(rows_a, acc_sh.at[dst_all.at[j0, 0]], add=True)
      cp_b.wait()
      pltpu.sync_copy(rows_b, acc_sh.at[dst_all.at[j0 + 1, 0]], add=True)
      return carry

    lax.fori_loop(0, j_max // UN, group, 0)
    plsc.subcore_barrier()

    pltpu.sync_copy(acc_sh.at[pl.ds(base, rows_per_tile)],
                    g_out.at[pl.ds(cid * n_pad + base, rows_per_tile)])

  return pl.kernel(
      body,
      out_type=jax.ShapeDtypeStruct((NC * n_pad, w), jnp.float32),
      mesh=mesh,
      scratch_types=[
          pltpu.VMEM_SHARED((n_pad, w), jnp.float32),
          pltpu.VMEM((j_max, 1, CH), jnp.int32),
          pltpu.VMEM((j_max, 1, CH), jnp.int32),
          pltpu.VMEM((CH, w), jnp.float32),
          pltpu.VMEM((CH, w), jnp.float32),
          pltpu.SemaphoreType.DMA,
          pltpu.SemaphoreType.DMA,
      ])


@functools.lru_cache(maxsize=None)
def _make_deg(n_pad: int, w: int, e_pad: int):
  assert e_pad % (CH * NW * UN) == 0 and n_pad % (NS * 8) == 0
  rows_per_tile = n_pad // NS
  j_max = e_pad // (CH * NW)

  mesh = plsc.VectorSubcoreMesh(
      core_axis_name="c", subcore_axis_name="s",
      num_cores=NC, num_subcores=NS)

  def body(dst_hbm, z_hbm, ones_hbm, d_out, deg_sh, dst_all, ones_v):
    cid = lax.axis_index("c")
    sid = lax.axis_index("s")
    wid = sid * NC + cid
    base = sid * rows_per_tile
    cbase = wid * j_max

    pltpu.sync_copy(dst_hbm.at[pl.ds(cbase, j_max)], dst_all)
    pltpu.sync_copy(z_hbm, deg_sh.at[pl.ds(base, rows_per_tile)])
    pltpu.sync_copy(ones_hbm, ones_v)
    plsc.subcore_barrier()

    def chunk(j, carry):
      pltpu.sync_copy(ones_v, deg_sh.at[dst_all.at[j, 0]], add=True)
      return carry

    lax.fori_loop(0, j_max, chunk, 0)
    plsc.subcore_barrier()

    pltpu.sync_copy(deg_sh.at[pl.ds(base, rows_per_tile)],
                    d_out.at[pl.ds(cid * n_pad + base, rows_per_tile)])

  return pl.kernel(
      body,
      out_type=jax.ShapeDtypeStruct((NC * n_pad, w), jnp.float32),
      mesh=mesh,
      scratch_types=[
          pltpu.VMEM_SHARED((n_pad, w), jnp.float32),
          pltpu.VMEM((j_max, 1, CH), jnp.int32),
          pltpu.VMEM((CH, w), jnp.float32),
      ])


_R = 256


def _mm1(x, w1s, b1s, w1n):
  n_pad, d = x.shape
  h = w1s.shape[1]

  def body(x_ref, ws_ref, bs_ref, wn_ref, s_ref, p_ref):
    xb = x_ref[...]
    s = jnp.dot(xb, ws_ref[...], preferred_element_type=jnp.float32)
    s_ref[...] = jnp.maximum(s + bs_ref[...], 0.0)
    p_ref[...] = jnp.dot(xb, wn_ref[...], preferred_element_type=jnp.float32)

  return pl.pallas_call(
      body,
      grid=(n_pad // _R,),
      in_specs=[
          pl.BlockSpec((_R, d), lambda i: (i, 0)),
          pl.BlockSpec((d, h), lambda i: (0, 0)),
          pl.BlockSpec((1, h), lambda i: (0, 0)),
          pl.BlockSpec((d, h), lambda i: (0, 0)),
      ],
      out_specs=[
          pl.BlockSpec((_R, h), lambda i: (i, 0)),
          pl.BlockSpec((_R, h), lambda i: (i, 0)),
      ],
      out_shape=[jax.ShapeDtypeStruct((n_pad, h), jnp.float32),
                 jax.ShapeDtypeStruct((n_pad, h), jnp.float32)],
  )(x, w1s, b1s.reshape(1, h), w1n)


def _agg_half(g0, g1, d0, d1, bn):
  g = g0 + g1
  deg = d0[:, :1] + d1[:, :1]
  agg = g / jnp.maximum(deg, 1.0)
  return jnp.maximum(agg + bn, 0.0)


def _mm2(s1, g0, g1, d0, d1, b1n, w2s, b2s, w2n):
  n_pad, h = s1.shape
  d2 = 2 * h

  def body(s1_ref, g0_ref, g1_ref, d0_ref, d1_ref, bn_ref,
           ws_ref, bs_ref, wn_ref, s_ref, p_ref):
    hn = _agg_half(g0_ref[...], g1_ref[...], d0_ref[...], d1_ref[...],
                   bn_ref[...])
    h1 = jnp.concatenate([s1_ref[...], hn], axis=1)
    s = jnp.dot(h1, ws_ref[...], preferred_element_type=jnp.float32)
    s_ref[...] = jnp.maximum(s + bs_ref[...], 0.0)
    p_ref[...] = jnp.dot(h1, wn_ref[...], preferred_element_type=jnp.float32)

  row = lambda i: (i, 0)
  fixed = lambda i: (0, 0)
  return pl.pallas_call(
      body,
      grid=(n_pad // _R,),
      in_specs=[
          pl.BlockSpec((_R, h), row),
          pl.BlockSpec((_R, h), row),
          pl.BlockSpec((_R, h), row),
          pl.BlockSpec((_R, h), row),
          pl.BlockSpec((_R, h), row),
          pl.BlockSpec((1, h), fixed),
          pl.BlockSpec((d2, h), fixed),
          pl.BlockSpec((1, h), fixed),
          pl.BlockSpec((d2, h), fixed),
      ],
      out_specs=[
          pl.BlockSpec((_R, h), row),
          pl.BlockSpec((_R, h), row),
      ],
      out_shape=[jax.ShapeDtypeStruct((n_pad, h), jnp.float32)] * 2,
  )(s1, g0, g1, d0, d1, b1n.reshape(1, h), w2s, b2s.reshape(1, h), w2n)


def _mm3(s2, g0, g1, d0, d1, b2n, wc_pad, bc_pad):
  n_pad, h = s2.shape
  d2 = 2 * h
  cp = wc_pad.shape[1]

  def body(s2_ref, g0_ref, g1_ref, d0_ref, d1_ref, bn_ref,
           wc_ref, bc_ref, o_ref):
    hn = _agg_half(g0_ref[...], g1_ref[...], d0_ref[...], d1_ref[...],
                   bn_ref[...])
    h2 = jnp.concatenate([s2_ref[...], hn], axis=1)
    norm = jnp.sqrt(jnp.sum(h2 * h2, axis=1, keepdims=True))
    h2 = h2 / jnp.maximum(norm, 1e-12)
    o = jnp.dot(h2, wc_ref[...], preferred_element_type=jnp.float32)
    o_ref[...] = o + bc_ref[...]

  row = lambda i: (i, 0)
  fixed = lambda i: (0, 0)
  return pl.pallas_call(
      body,
      grid=(n_pad // _R,),
      in_specs=[
          pl.BlockSpec((_R, h), row),
          pl.BlockSpec((_R, h), row),
          pl.BlockSpec((_R, h), row),
          pl.BlockSpec((_R, h), row),
          pl.BlockSpec((_R, h), row),
          pl.BlockSpec((1, h), fixed),
          pl.BlockSpec((d2, cp), fixed),
          pl.BlockSpec((1, cp), fixed),
      ],
      out_specs=pl.BlockSpec((_R, cp), row),
      out_shape=jax.ShapeDtypeStruct((n_pad, cp), jnp.float32),
  )(s2, g0, g1, d0, d1, b2n.reshape(1, h), wc_pad, bc_pad.reshape(1, cp))


def kernel(x, edge_index, W1s, b1s, W1n, b1n, W2s, b2s, W2n, b2n, Wc, bc):
  n, d = x.shape
  h = W1s.shape[1]
  e = edge_index.shape[1]
  c = Wc.shape[1]
  align = max(NS * 8, _R)
  n_pad = ((n + align - 1) // align) * align
  rows_per_tile = n_pad // NS
  cp = 128
  eu = CH * NW * UN
  e_pad = ((e + eu - 1) // eu) * eu

  src = jnp.pad(edge_index[0], (0, e_pad - e)).reshape(e_pad // CH, 1, CH)
  dst = jnp.pad(edge_index[1], (0, e_pad - e),
                constant_values=n).reshape(e_pad // CH, 1, CH)
  x_pad = jnp.pad(x, ((0, n_pad - n), (0, 0)))
  wc_pad = jnp.pad(Wc, ((0, 0), (0, cp - c)))
  bc_pad = jnp.pad(bc, (0, cp - c))
  zeros_h = jnp.zeros((rows_per_tile, h), jnp.float32)
  ones = jnp.ones((CH, h), jnp.float32)

  s1, p1 = _mm1(x_pad, W1s, b1s, W1n)

  degp = _make_deg(n_pad, h, e_pad)(dst, zeros_h, ones)
  dega, degb = degp[:n_pad], degp[n_pad:]

  g1p = _make_seg_sum(n_pad, h, e_pad)(p1, src, dst, zeros_h)
  g1a, g1b = g1p[:n_pad], g1p[n_pad:]

  s2, p2 = _mm2(s1, g1a, g1b, dega, degb, b1n, W2s, b2s, W2n)

  g2p = _make_seg_sum(n_pad, h, e_pad)(p2, src, dst, zeros_h)
  g2a, g2b = g2p[:n_pad], g2p[n_pad:]

  out = _mm3(s2, g2a, g2b, dega, degb, b2n, wc_pad, bc_pad)
  return out[:n, :c]


# speedup vs baseline: 1.1736x; 1.1736x over previous
import functools

import jax
import jax.numpy as jnp
from jax import lax
from jax.experimental import pallas as pl
from jax.experimental.pallas import tpu as pltpu
from jax.experimental.pallas import tpu_sc as plsc

NC = 2
NS = 16
NW = NC * NS
CH = 128
UN = 2


@functools.lru_cache(maxsize=None)
def _make_seg_sum(n_pad: int, w: int, e_pad: int):
  assert e_pad % (CH * NW * UN) == 0 and n_pad % (NS * 8) == 0
  rows_per_tile = n_pad // NS
  j_max = e_pad // (CH * NW)

  mesh = plsc.VectorSubcoreMesh(
      core_axis_name="c", subcore_axis_name="s",
      num_cores=NC, num_subcores=NS)

  def body(p_hbm, src_hbm, dst_hbm, z_hbm, g_out, acc_sh, *bufs):
    src_vs = bufs[0 * UN:1 * UN]
    dst_vs = bufs[1 * UN:2 * UN]
    rows_vs = bufs[2 * UN:3 * UN]
    isems = bufs[3 * UN:4 * UN]
    dsems = bufs[4 * UN:5 * UN]
    gsems = bufs[5 * UN:6 * UN]
    cid = lax.axis_index("c")
    sid = lax.axis_index("s")
    wid = sid * NC + cid
    base = sid * rows_per_tile

    def fire_idx(j0):
      for b in range(UN):
        off = (wid + (j0 + b) * NW) * CH
        pltpu.async_copy(src_hbm.at[pl.ds(off, CH)], src_vs[b], isems[b])
        pltpu.async_copy(dst_hbm.at[pl.ds(off, CH)], dst_vs[b], dsems[b])

    def wait_idx(b):
      pltpu.make_async_copy(src_hbm.at[pl.ds(0, CH)], src_vs[b],
                            isems[b]).wait()
      pltpu.make_async_copy(dst_hbm.at[pl.ds(0, CH)], dst_vs[b],
                            dsems[b]).wait()

    fire_idx(0)
    pltpu.sync_copy(z_hbm, acc_sh.at[pl.ds(base, rows_per_tile)])
    plsc.subcore_barrier()

    def group(g, carry):
      j0 = g * UN
      g_cp = []
      for b in range(UN):
        wait_idx(b)
        g_cp.append(pltpu.async_copy(p_hbm.at[src_vs[b]], rows_vs[b], gsems[b]))
      for b in range(UN):
        g_cp[b].wait()
        pltpu.sync_copy(rows_vs[b], acc_sh.at[dst_vs[b]], add=True)
        off = (wid + (j0 + UN + b) * NW) * CH
        pltpu.async_copy(src_hbm.at[pl.ds(off, CH)], src_vs[b], isems[b])
        pltpu.async_copy(dst_hbm.at[pl.ds(off, CH)], dst_vs[b], dsems[b])
      return carry

    lax.fori_loop(0, j_max // UN, group, 0)
    for b in range(UN):
      wait_idx(b)
    plsc.subcore_barrier()

    pltpu.sync_copy(acc_sh.at[pl.ds(base, rows_per_tile)],
                    g_out.at[pl.ds(cid * n_pad + base, rows_per_tile)])

  return pl.kernel(
      body,
      out_type=jax.ShapeDtypeStruct((NC * n_pad, w), jnp.float32),
      mesh=mesh,
      scratch_types=(
          [pltpu.VMEM_SHARED((n_pad, w), jnp.float32)] +
          [pltpu.VMEM((CH,), jnp.int32) for _ in range(UN)] +
          [pltpu.VMEM((CH,), jnp.int32) for _ in range(UN)] +
          [pltpu.VMEM((CH, w), jnp.float32) for _ in range(UN)] +
          [pltpu.SemaphoreType.DMA for _ in range(3 * UN)]))


@functools.lru_cache(maxsize=None)
def _make_deg(n_pad: int, w: int, e_pad: int):
  assert e_pad % (CH * NW * UN) == 0 and n_pad % (NS * 8) == 0
  rows_per_tile = n_pad // NS
  j_max = e_pad // (CH * NW)

  mesh = plsc.VectorSubcoreMesh(
      core_axis_name="c", subcore_axis_name="s",
      num_cores=NC, num_subcores=NS)

  def body(dst_hbm, z_hbm, ones_hbm, d_out, deg_sh, dst_all, ones_v):
    cid = lax.axis_index("c")
    sid = lax.axis_index("s")
    wid = sid * NC + cid
    base = sid * rows_per_tile
    cbase = wid * j_max

    pltpu.sync_copy(dst_hbm.at[pl.ds(cbase, j_max)], dst_all)
    pltpu.sync_copy(z_hbm, deg_sh.at[pl.ds(base, rows_per_tile)])
    pltpu.sync_copy(ones_hbm, ones_v)
    plsc.subcore_barrier()

    def chunk(j, carry):
      pltpu.sync_copy(ones_v, deg_sh.at[dst_all.at[j, 0]], add=True)
      return carry

    lax.fori_loop(0, j_max, chunk, 0)
    plsc.subcore_barrier()

    pltpu.sync_copy(deg_sh.at[pl.ds(base, rows_per_tile)],
                    d_out.at[pl.ds(cid * n_pad + base, rows_per_tile)])

  return pl.kernel(
      body,
      out_type=jax.ShapeDtypeStruct((NC * n_pad, w), jnp.float32),
      mesh=mesh,
      scratch_types=[
          pltpu.VMEM_SHARED((n_pad, w), jnp.float32),
          pltpu.VMEM((j_max, 1, CH), jnp.int32),
          pltpu.VMEM((CH, w), jnp.float32),
      ])


_R = 256


def _mm1(x, w1s, b1s, w1n):
  n_pad, d = x.shape
  h = w1s.shape[1]

  def body(x_ref, ws_ref, bs_ref, wn_ref, s_ref, p_ref):
    xb = x_ref[...]
    s = jnp.dot(xb, ws_ref[...], preferred_element_type=jnp.float32)
    s_ref[...] = jnp.maximum(s + bs_ref[...], 0.0)
    p_ref[...] = jnp.dot(xb, wn_ref[...], preferred_element_type=jnp.float32)

  return pl.pallas_call(
      body,
      grid=(n_pad // _R,),
      in_specs=[
          pl.BlockSpec((_R, d), lambda i: (i, 0)),
          pl.BlockSpec((d, h), lambda i: (0, 0)),
          pl.BlockSpec((1, h), lambda i: (0, 0)),
          pl.BlockSpec((d, h), lambda i: (0, 0)),
      ],
      out_specs=[
          pl.BlockSpec((_R, h), lambda i: (i, 0)),
          pl.BlockSpec((_R, h), lambda i: (i, 0)),
      ],
      out_shape=[jax.ShapeDtypeStruct((n_pad, h), jnp.float32),
                 jax.ShapeDtypeStruct((n_pad, h), jnp.float32)],
  )(x, w1s, b1s.reshape(1, h), w1n)


def _agg_half(g0, g1, d0, d1, bn):
  g = g0 + g1
  deg = d0[:, :1] + d1[:, :1]
  agg = g / jnp.maximum(deg, 1.0)
  return jnp.maximum(agg + bn, 0.0)


def _mm2(s1, g0, g1, d0, d1, b1n, w2s, b2s, w2n):
  n_pad, h = s1.shape
  d2 = 2 * h

  def body(s1_ref, g0_ref, g1_ref, d0_ref, d1_ref, bn_ref,
           ws_ref, bs_ref, wn_ref, s_ref, p_ref):
    hn = _agg_half(g0_ref[...], g1_ref[...], d0_ref[...], d1_ref[...],
                   bn_ref[...])
    h1 = jnp.concatenate([s1_ref[...], hn], axis=1)
    s = jnp.dot(h1, ws_ref[...], preferred_element_type=jnp.float32)
    s_ref[...] = jnp.maximum(s + bs_ref[...], 0.0)
    p_ref[...] = jnp.dot(h1, wn_ref[...], preferred_element_type=jnp.float32)

  row = lambda i: (i, 0)
  fixed = lambda i: (0, 0)
  return pl.pallas_call(
      body,
      grid=(n_pad // _R,),
      in_specs=[
          pl.BlockSpec((_R, h), row),
          pl.BlockSpec((_R, h), row),
          pl.BlockSpec((_R, h), row),
          pl.BlockSpec((_R, h), row),
          pl.BlockSpec((_R, h), row),
          pl.BlockSpec((1, h), fixed),
          pl.BlockSpec((d2, h), fixed),
          pl.BlockSpec((1, h), fixed),
          pl.BlockSpec((d2, h), fixed),
      ],
      out_specs=[
          pl.BlockSpec((_R, h), row),
          pl.BlockSpec((_R, h), row),
      ],
      out_shape=[jax.ShapeDtypeStruct((n_pad, h), jnp.float32)] * 2,
  )(s1, g0, g1, d0, d1, b1n.reshape(1, h), w2s, b2s.reshape(1, h), w2n)


def _mm3(s2, g0, g1, d0, d1, b2n, wc_pad, bc_pad):
  n_pad, h = s2.shape
  d2 = 2 * h
  cp = wc_pad.shape[1]

  def body(s2_ref, g0_ref, g1_ref, d0_ref, d1_ref, bn_ref,
           wc_ref, bc_ref, o_ref):
    hn = _agg_half(g0_ref[...], g1_ref[...], d0_ref[...], d1_ref[...],
                   bn_ref[...])
    h2 = jnp.concatenate([s2_ref[...], hn], axis=1)
    norm = jnp.sqrt(jnp.sum(h2 * h2, axis=1, keepdims=True))
    h2 = h2 / jnp.maximum(norm, 1e-12)
    o = jnp.dot(h2, wc_ref[...], preferred_element_type=jnp.float32)
    o_ref[...] = o + bc_ref[...]

  row = lambda i: (i, 0)
  fixed = lambda i: (0, 0)
  return pl.pallas_call(
      body,
      grid=(n_pad // _R,),
      in_specs=[
          pl.BlockSpec((_R, h), row),
          pl.BlockSpec((_R, h), row),
          pl.BlockSpec((_R, h), row),
          pl.BlockSpec((_R, h), row),
          pl.BlockSpec((_R, h), row),
          pl.BlockSpec((1, h), fixed),
          pl.BlockSpec((d2, cp), fixed),
          pl.BlockSpec((1, cp), fixed),
      ],
      out_specs=pl.BlockSpec((_R, cp), row),
      out_shape=jax.ShapeDtypeStruct((n_pad, cp), jnp.float32),
  )(s2, g0, g1, d0, d1, b2n.reshape(1, h), wc_pad, bc_pad.reshape(1, cp))


def kernel(x, edge_index, W1s, b1s, W1n, b1n, W2s, b2s, W2n, b2n, Wc, bc):
  n, d = x.shape
  h = W1s.shape[1]
  e = edge_index.shape[1]
  c = Wc.shape[1]
  align = max(NS * 8, _R)
  n_pad = ((n + align - 1) // align) * align
  rows_per_tile = n_pad // NS
  cp = 128
  eu = CH * NW * UN
  e_pad = ((e + eu - 1) // eu) * eu

  e_alloc = e_pad + UN * NW * CH
  src = jnp.pad(edge_index[0], (0, e_alloc - e))
  dst = jnp.pad(edge_index[1], (0, e_alloc - e), constant_values=n)
  dst3 = dst[:e_pad].reshape(e_pad // CH, 1, CH)
  x_pad = jnp.pad(x, ((0, n_pad - n), (0, 0)))
  wc_pad = jnp.pad(Wc, ((0, 0), (0, cp - c)))
  bc_pad = jnp.pad(bc, (0, cp - c))
  zeros_h = jnp.zeros((rows_per_tile, h), jnp.float32)
  ones = jnp.ones((CH, h), jnp.float32)

  s1, p1 = _mm1(x_pad, W1s, b1s, W1n)

  degp = _make_deg(n_pad, h, e_pad)(dst3, zeros_h, ones)
  dega, degb = degp[:n_pad], degp[n_pad:]

  g1p = _make_seg_sum(n_pad, h, e_pad)(p1, src, dst, zeros_h)
  g1a, g1b = g1p[:n_pad], g1p[n_pad:]

  s2, p2 = _mm2(s1, g1a, g1b, dega, degb, b1n, W2s, b2s, W2n)

  g2p = _make_seg_sum(n_pad, h, e_pad)(p2, src, dst, zeros_h)
  g2a, g2b = g2p[:n_pad], g2p[n_pad:]

  out = _mm3(s2, g2a, g2b, dega, degb, b2n, wc_pad, bc_pad)
  return out[:n, :c]
